# baseline (speedup 1.0000x reference)
import functools
from functools import partial

import jax
import jax.numpy as jnp
from jax import lax
from jax.experimental import pallas as pl
from jax.experimental.pallas import tpu as pltpu
from jax.experimental.pallas import tpu_sc as plsc

N = 10000
E = 320000
D = 128
B = 64
NC = 2
NS = 16
NW = NC * NS

EP = 331776
EPT = EP // NW
CHUNKS = EPT // 128
NPAD = 10240

_f32 = jnp.float32


def _leaky(x):
    return jnp.where(x > 0, x, 0.01 * x)


def _hist_body(row_hbm, out_hbm, idxt_v, hist_v):
    c = lax.axis_index("c")
    s = lax.axis_index("s")
    wid = s * NC + c
    ept = E // NW

    def zbody(i, _):
        hist_v[pl.ds(i * 16, 16)] = jnp.zeros((16,), _f32)
        return 0
    lax.fori_loop(0, N // 16, zbody, 0)

    pltpu.sync_copy(row_hbm.at[pl.ds(wid * ept, ept)], idxt_v)

    ones16 = jnp.ones((16,), _f32)

    def grp(i, _):
        idx16 = idxt_v[pl.ds(i * 16, 16)]
        plsc.addupdate_scatter(hist_v, [idx16], ones16)
        return 0
    lax.fori_loop(0, ept // 16, grp, 0)

    pltpu.sync_copy(hist_v, out_hbm.at[wid])


_SC_PARAMS = pltpu.CompilerParams(needs_layout_passes=False)

_hist_kernel = pl.kernel(
    _hist_body,
    out_type=jax.ShapeDtypeStruct((NW, N), _f32),
    mesh=plsc.VectorSubcoreMesh(core_axis_name="c", subcore_axis_name="s"),
    compiler_params=_SC_PARAMS,
    scratch_types=[
        pltpu.VMEM((E // NW,), jnp.int32),
        pltpu.VMEM((N,), _f32),
    ],
)


def _dinv_body(h_ref, o_ref):
    s = jnp.sum(h_ref[...], axis=0, keepdims=True)
    o_ref[...] = lax.rsqrt(s + 1.0)


def _dinv_kernel(hists):
    return pl.pallas_call(
        _dinv_body,
        out_shape=jax.ShapeDtypeStruct((1, N), _f32),
    )(hists)


CH = 128
NCH = EPT // CH


def _msg_body(g_hbm, ieab, colb, zeros_hbm, out_hbm, colt, iea0, iea1,
              rows0, rows1, acc, semi0, semi1, semg0, semg1, sems0, sems1):
    c = lax.axis_index("c")
    s = lax.axis_index("s")
    wid = s * NC + c
    cbase = wid * NCH

    def stage(ci, ieabuf, sem):
        pltpu.async_copy(ieab.at[cbase + ci], ieabuf, sem)

    def wait_stage(ieabuf, sem):
        pltpu.make_async_copy(ieab.at[0], ieabuf, sem).wait()

    def fire_gather(ieabuf, buf, sem):
        pltpu.async_copy(g_hbm.at[ieabuf.at[0]], buf, sem)

    def wait_dma(buf, sem):
        pltpu.make_async_copy(zeros_hbm, buf, sem).wait()

    def fire_scatter(ci, buf, sem):
        pltpu.async_copy(buf, acc.at[colt.at[ci]], sem, add=True)

    row1sel = jnp.ones((16,), jnp.int32)

    def scale(ieabuf, buf):
        def ebody(e, _):
            eidx = jnp.broadcast_to(e.astype(jnp.int32), (16,))
            cv = plsc.bitcast(plsc.load_gather(ieabuf, [row1sel, eidx]),
                              _f32)
            for j in range(8):
                sl = pl.ds(j * 16, 16)
                buf[e, sl] = buf[e, sl] * cv
            return 0
        lax.fori_loop(0, CH, ebody, 0, unroll=4)

    stage(0, iea0, semi0)
    stage(1, iea1, semi1)
    pltpu.sync_copy(colb.at[wid], colt)
    for k in range(5):
        pltpu.sync_copy(zeros_hbm, acc.at[pl.ds(s * 640 + k * 128, 128)])
    plsc.subcore_barrier()

    wait_stage(iea0, semi0)
    fire_gather(iea0, rows0, semg0)

    def pair(c0, first):
        c1 = c0 + 1
        wait_stage(iea1, semi1)
        if not first:
            wait_dma(rows1, sems1)
        fire_gather(iea1, rows1, semg1)
        wait_dma(rows0, semg0)
        scale(iea0, rows0)
        fire_scatter(c0, rows0, sems0)
        stage(c0 + 2, iea0, semi0)
        wait_dma(rows1, semg1)
        scale(iea1, rows1)
        fire_scatter(c1, rows1, sems1)
        stage(jnp.minimum(c0 + 3, NCH - 1), iea1, semi1)
        wait_stage(iea0, semi0)
        wait_dma(rows0, sems0)
        fire_gather(iea0, rows0, semg0)

    pair(0, True)

    def pbody(i2, _):
        pair(2 * i2, False)
        return 0
    lax.fori_loop(1, (NCH - 1) // 2, pbody, 0)

    wait_stage(iea1, semi1)
    wait_dma(rows0, semg0)
    scale(iea0, rows0)
    fire_scatter(NCH - 1, rows0, sems0)
    wait_dma(rows1, sems1)
    wait_dma(rows0, sems0)

    plsc.subcore_barrier()
    for k in range(5):
        off = s * 640 + k * 128
        pltpu.sync_copy(acc.at[pl.ds(off, 128)],
                        out_hbm.at[c, pl.ds(off, 128)])


_msg_kernel = pl.kernel(
    _msg_body,
    out_type=jax.ShapeDtypeStruct((NC, NPAD, D), _f32),
    mesh=plsc.VectorSubcoreMesh(core_axis_name="c", subcore_axis_name="s"),
    compiler_params=_SC_PARAMS,
    scratch_types=(
        [pltpu.VMEM((NCH, CH), jnp.int32)]
        + [pltpu.VMEM((2, CH), jnp.int32) for _ in range(2)]
        + [pltpu.VMEM((CH, D), _f32) for _ in range(2)]
        + [pltpu.VMEM_SHARED((NPAD, D), _f32)]
        + [pltpu.SemaphoreType.DMA for _ in range(6)]
    ),
)


RB = 400
NBLK = N // RB


def _fused0_body(x_ref, dv_ref, w1_ref, b1_ref, g_ref, bt_ref, w2_ref,
                 b2_ref, o_ref, h1s, sts):
    ph = pl.program_id(0)
    i = pl.program_id(1)

    @pl.when(ph == 0)
    def _():
        @pl.when(i == 0)
        def _():
            sts[...] = jnp.zeros((8, D), _f32)

        a = jnp.dot(x_ref[...], w1_ref[...],
                    preferred_element_type=_f32) + b1_ref[...]
        h1s[pl.ds(i * RB, RB), :] = a
        sts[0:1, :] += jnp.sum(a, axis=0, keepdims=True)
        sts[1:2, :] += jnp.sum(a * a, axis=0, keepdims=True)
        o_ref[...] = a

    @pl.when(ph == 1)
    def _():
        m = sts[0:1, :] * (1.0 / N)
        ex2 = sts[1:2, :] * (1.0 / N)
        var = ex2 - m * m
        rstd = lax.rsqrt(var + 1e-5)
        aa = g_ref[...] * rstd
        cc = bt_ref[...] - m * aa
        t = _leaky(h1s[pl.ds(i * RB, RB), :] * aa + cc)
        o_ref[...] = dv_ref[...] * (
            jnp.dot(t, w2_ref[...], preferred_element_type=_f32)
            + b2_ref[...])


def _fused0(x, dinv_col, w1, b1, gbn, bt, w2, b2):
    return pl.pallas_call(
        _fused0_body,
        grid=(2, NBLK),
        in_specs=[
            pl.BlockSpec((RB, D), lambda p, i: (i, 0)),
            pl.BlockSpec((RB, 1), lambda p, i: (i, 0)),
            pl.BlockSpec((D, D), lambda p, i: (0, 0)),
            pl.BlockSpec((1, D), lambda p, i: (0, 0)),
            pl.BlockSpec((1, D), lambda p, i: (0, 0)),
            pl.BlockSpec((1, D), lambda p, i: (0, 0)),
            pl.BlockSpec((D, D), lambda p, i: (0, 0)),
            pl.BlockSpec((1, D), lambda p, i: (0, 0)),
        ],
        out_specs=pl.BlockSpec((RB, D), lambda p, i: (i, 0)),
        out_shape=jax.ShapeDtypeStruct((N, D), _f32),
        scratch_shapes=[pltpu.VMEM((N, D), _f32), pltpu.VMEM((8, D), _f32)],
    )(x, dinv_col, w1, b1.reshape(1, D), gbn.reshape(1, D),
      bt.reshape(1, D), w2, b2.reshape(1, D))


def _fused1_body(p0_ref, p1_ref, dv_ref, w1_ref, b1_ref, g_ref, bt_ref,
                 w2_ref, b2_ref, o_ref, h1s, sts):
    ph = pl.program_id(0)
    i = pl.program_id(1)

    @pl.when(ph == 0)
    def _():
        @pl.when(i == 0)
        def _():
            sts[...] = jnp.zeros((8, D), _f32)

        x = _leaky(dv_ref[...] * (p0_ref[0] + p1_ref[0]))
        a = jnp.dot(x, w1_ref[...],
                    preferred_element_type=_f32) + b1_ref[...]
        h1s[pl.ds(i * RB, RB), :] = a
        sts[0:1, :] += jnp.sum(a, axis=0, keepdims=True)
        sts[1:2, :] += jnp.sum(a * a, axis=0, keepdims=True)
        o_ref[...] = a

    @pl.when(ph == 1)
    def _():
        m = sts[0:1, :] * (1.0 / N)
        ex2 = sts[1:2, :] * (1.0 / N)
        var = ex2 - m * m
        rstd = lax.rsqrt(var + 1e-5)
        aa = g_ref[...] * rstd
        cc = bt_ref[...] - m * aa
        t = _leaky(h1s[pl.ds(i * RB, RB), :] * aa + cc)
        o_ref[...] = dv_ref[...] * (
            jnp.dot(t, w2_ref[...], preferred_element_type=_f32)
            + b2_ref[...])


def _fused1(parts, dinv_col, w1, b1, gbn, bt, w2, b2):
    return pl.pallas_call(
        _fused1_body,
        grid=(2, NBLK),
        in_specs=[
            pl.BlockSpec((1, RB, D), lambda p, i: (0, i, 0)),
            pl.BlockSpec((1, RB, D), lambda p, i: (1, i, 0)),
            pl.BlockSpec((RB, 1), lambda p, i: (i, 0)),
            pl.BlockSpec((D, D), lambda p, i: (0, 0)),
            pl.BlockSpec((1, D), lambda p, i: (0, 0)),
            pl.BlockSpec((1, D), lambda p, i: (0, 0)),
            pl.BlockSpec((1, D), lambda p, i: (0, 0)),
            pl.BlockSpec((D, D), lambda p, i: (0, 0)),
            pl.BlockSpec((1, D), lambda p, i: (0, 0)),
        ],
        out_specs=pl.BlockSpec((RB, D), lambda p, i: (i, 0)),
        out_shape=jax.ShapeDtypeStruct((N, D), _f32),
        scratch_shapes=[pltpu.VMEM((N, D), _f32), pltpu.VMEM((8, D), _f32)],
    )(parts, parts, dinv_col, w1, b1.reshape(1, D), gbn.reshape(1, D),
      bt.reshape(1, D), w2, b2.reshape(1, D))


def _final_body(q0_ref, q1_ref, dv_ref, bcol_ref, blo_ref, bhi_ref, en_ref,
                ew_ref, eb_ref, eg_ref, ebt_ref, wa_ref, wb_ref, lb1_ref,
                lg_ref, lbt_ref, lw2_ref, lb2_ref, o_ref, pooled):
    i = pl.program_id(0)

    @pl.when(i == 0)
    def _():
        pooled[...] = jnp.full((B, D), -jnp.inf, _f32)

    h = _leaky(dv_ref[...] * (q0_ref[0] + q1_ref[0]))
    bblk = bcol_ref[0]
    blo = blo_ref[i]
    bhi = bhi_ref[i]

    def seg(b, _):
        mask = bblk == b
        vals = jnp.where(mask, h, -jnp.inf)
        m = jnp.max(vals, axis=0, keepdims=True)
        pooled[pl.ds(b, 1)] = jnp.maximum(pooled[pl.ds(b, 1)], m)
        return 0
    lax.fori_loop(blo, bhi + 1, seg, 0)

    @pl.when(i == NBLK - 1)
    def _():
        e0 = jnp.dot(en_ref[...], ew_ref[...], preferred_element_type=_f32) \
            + eb_ref[...]
        m = jnp.mean(e0, axis=0, keepdims=True)
        v = jnp.mean((e0 - m) * (e0 - m), axis=0, keepdims=True)
        e1 = _leaky((e0 - m) * lax.rsqrt(v + 1e-5) * eg_ref[...] + ebt_ref[...])
        z = jnp.dot(pooled[...], wa_ref[...], preferred_element_type=_f32) \
            + jnp.dot(e1, wb_ref[...], preferred_element_type=_f32) \
            + lb1_ref[...]
        zm = jnp.mean(z, axis=0, keepdims=True)
        zv = jnp.mean((z - zm) * (z - zm), axis=0, keepdims=True)
        z2 = _leaky((z - zm) * lax.rsqrt(zv + 1e-5) * lg_ref[...] + lbt_ref[...])
        y = jnp.dot(z2, lw2_ref[...], preferred_element_type=_f32) + lb2_ref[...]
        o_ref[...] = -jnp.maximum(y, 0.0)


def _final(qparts, dinv_col, bcol, blo, bhi, energy, p):
    wa = p['l_w1'][:D, :]
    wb = p['l_w1'][D:, :]
    return pl.pallas_call(
        _final_body,
        grid=(NBLK,),
        in_specs=[
            pl.BlockSpec((1, RB, D), lambda i: (0, i, 0)),
            pl.BlockSpec((1, RB, D), lambda i: (1, i, 0)),
            pl.BlockSpec((RB, 1), lambda i: (i, 0)),
            pl.BlockSpec((1, RB, 1), lambda i: (i, 0, 0)),
            pl.BlockSpec(memory_space=pltpu.SMEM),
            pl.BlockSpec(memory_space=pltpu.SMEM),
            pl.BlockSpec((B, 21), lambda i: (0, 0)),
            pl.BlockSpec((21, 8), lambda i: (0, 0)),
            pl.BlockSpec((1, 8), lambda i: (0, 0)),
            pl.BlockSpec((1, 8), lambda i: (0, 0)),
            pl.BlockSpec((1, 8), lambda i: (0, 0)),
            pl.BlockSpec((D, D), lambda i: (0, 0)),
            pl.BlockSpec((8, D), lambda i: (0, 0)),
            pl.BlockSpec((1, D), lambda i: (0, 0)),
            pl.BlockSpec((1, D), lambda i: (0, 0)),
            pl.BlockSpec((1, D), lambda i: (0, 0)),
            pl.BlockSpec((D, 1), lambda i: (0, 0)),
            pl.BlockSpec((1, 1), lambda i: (0, 0)),
        ],
        out_specs=pl.BlockSpec((B, 1), lambda i: (0, 0)),
        out_shape=jax.ShapeDtypeStruct((B, 1), _f32),
        scratch_shapes=[pltpu.VMEM((B, D), _f32)],
    )(qparts, qparts, dinv_col, bcol, blo, bhi, energy,
      p['e_w'], p['e_b'].reshape(1, 8), p['e_g'].reshape(1, 8),
      p['e_bt'].reshape(1, 8), wa, wb, p['l_b1'].reshape(1, D),
      p['l_g'].reshape(1, D), p['l_bt'].reshape(1, D), p['l_w2'],
      p['l_b2'].reshape(1, 1))


def kernel(x, edge_index, edge_attr, batch, energy, params):
    p = params
    pad = EP - E - N
    loop = jnp.arange(N, dtype=jnp.int32)
    rowp = jnp.concatenate(
        [edge_index[0], loop, jnp.zeros((pad,), jnp.int32)]).reshape(NW * NCH, CH)
    colp = jnp.concatenate(
        [edge_index[1], loop, jnp.zeros((pad,), jnp.int32)]).reshape(NW, NCH, CH)
    eabits = lax.bitcast_convert_type(
        jnp.concatenate([edge_attr, jnp.ones((N,), _f32),
                         jnp.zeros((pad,), _f32)]).reshape(NW * NCH, CH),
        jnp.int32)
    ieab = jnp.stack([rowp, eabits], axis=1)

    hists = _hist_kernel(edge_index[0])
    dinv_col = _dinv_kernel(hists).reshape(N, 1)

    zeros128 = jnp.zeros((128, D), _f32)

    g0 = _fused0(x, dinv_col, p['m0_w1'], p['m0_b1'], p['m0_g'], p['m0_bt'],
                 p['m0_w2'], p['m0_b2'])
    pparts = _msg_kernel(g0, ieab, colp, zeros128)

    g1 = _fused1(pparts, dinv_col, p['m1_w1'], p['m1_b1'],
                 p['m1_g'], p['m1_bt'], p['m1_w2'], p['m1_b2'])
    qparts = _msg_kernel(g1, ieab, colp, zeros128)

    bcol = batch.reshape(NBLK, RB, 1)
    b2d = batch.reshape(NBLK, RB)
    blo = b2d[:, 0]
    bhi = b2d[:, RB - 1]
    return _final(qparts, dinv_col, bcol, blo, bhi, energy, p)


# speedup vs baseline: 1.0053x; 1.0053x over previous
import functools
from functools import partial

import jax
import jax.numpy as jnp
from jax import lax
from jax.experimental import pallas as pl
from jax.experimental.pallas import tpu as pltpu
from jax.experimental.pallas import tpu_sc as plsc

N = 10000
E = 320000
D = 128
B = 64
NC = 2
NS = 16
NW = NC * NS

EP = 331776
EPT = EP // NW
CHUNKS = EPT // 128
NPAD = 10240

_f32 = jnp.float32


def _leaky(x):
    return jnp.where(x > 0, x, 0.01 * x)


def _hist_body(row_hbm, out_hbm, idxt_v, hist_v):
    c = lax.axis_index("c")
    s = lax.axis_index("s")
    wid = s * NC + c
    ept = E // NW

    def zbody(i, _):
        hist_v[pl.ds(i * 16, 16)] = jnp.zeros((16,), _f32)
        return 0
    lax.fori_loop(0, N // 16, zbody, 0)

    pltpu.sync_copy(row_hbm.at[pl.ds(wid * ept, ept)], idxt_v)

    ones16 = jnp.ones((16,), _f32)

    def grp(i, _):
        idx16 = idxt_v[pl.ds(i * 16, 16)]
        plsc.addupdate_scatter(hist_v, [idx16], ones16)
        return 0
    lax.fori_loop(0, ept // 16, grp, 0)

    pltpu.sync_copy(hist_v, out_hbm.at[wid])


_SC_PARAMS = pltpu.CompilerParams(needs_layout_passes=False)

_hist_kernel = pl.kernel(
    _hist_body,
    out_type=jax.ShapeDtypeStruct((NW, N), _f32),
    mesh=plsc.VectorSubcoreMesh(core_axis_name="c", subcore_axis_name="s"),
    compiler_params=_SC_PARAMS,
    scratch_types=[
        pltpu.VMEM((E // NW,), jnp.int32),
        pltpu.VMEM((N,), _f32),
    ],
)


def _dinv_body(h_ref, o_ref):
    s = jnp.sum(h_ref[...], axis=0, keepdims=True)
    o_ref[...] = lax.rsqrt(s + 1.0)


def _dinv_kernel(hists):
    return pl.pallas_call(
        _dinv_body,
        out_shape=jax.ShapeDtypeStruct((1, N), _f32),
    )(hists)


CH = 128
NCH = EPT // CH


def _msg_body(g_hbm, ieab, colb, zeros_hbm, out_hbm, colt, iea0, iea1,
              rows0, rows1, acc, semi0, semi1, semg0, semg1, sems0, sems1):
    c = lax.axis_index("c")
    s = lax.axis_index("s")
    wid = s * NC + c
    cbase = wid * NCH

    def stage(ci, ieabuf, sem):
        pltpu.async_copy(ieab.at[cbase + ci], ieabuf, sem)

    def wait_stage(ieabuf, sem):
        pltpu.make_async_copy(ieab.at[0], ieabuf, sem).wait()

    def fire_gather(ieabuf, buf, sem):
        pltpu.async_copy(g_hbm.at[ieabuf.at[0]], buf, sem)

    def wait_dma(buf, sem):
        pltpu.make_async_copy(zeros_hbm, buf, sem).wait()

    def fire_scatter(ci, buf, sem):
        pltpu.async_copy(buf, acc.at[colt.at[ci]], sem, add=True)

    row1sel = jnp.ones((16,), jnp.int32)

    def scale(ieabuf, buf):
        def ebody(e, _):
            eidx = jnp.broadcast_to(e.astype(jnp.int32), (16,))
            cv = plsc.bitcast(plsc.load_gather(ieabuf, [row1sel, eidx]),
                              _f32)
            for j in range(8):
                sl = pl.ds(j * 16, 16)
                buf[e, sl] = buf[e, sl] * cv
            return 0
        lax.fori_loop(0, CH, ebody, 0, unroll=4)

    stage(0, iea0, semi0)
    stage(1, iea1, semi1)
    pltpu.sync_copy(colb.at[wid], colt)
    for k in range(5):
        pltpu.sync_copy(zeros_hbm, acc.at[pl.ds(s * 640 + k * 128, 128)])
    plsc.subcore_barrier()

    wait_stage(iea0, semi0)
    fire_gather(iea0, rows0, semg0)

    def pair(c0, first):
        c1 = c0 + 1
        wait_stage(iea1, semi1)
        if not first:
            wait_dma(rows1, sems1)
        fire_gather(iea1, rows1, semg1)
        wait_dma(rows0, semg0)
        scale(iea0, rows0)
        fire_scatter(c0, rows0, sems0)
        stage(c0 + 2, iea0, semi0)
        wait_dma(rows1, semg1)
        scale(iea1, rows1)
        fire_scatter(c1, rows1, sems1)
        stage(jnp.minimum(c0 + 3, NCH - 1), iea1, semi1)
        wait_stage(iea0, semi0)
        wait_dma(rows0, sems0)
        fire_gather(iea0, rows0, semg0)

    pair(0, True)

    def pbody(i2, _):
        pair(2 * i2, False)
        return 0
    lax.fori_loop(1, (NCH - 1) // 2, pbody, 0)

    wait_stage(iea1, semi1)
    wait_dma(rows0, semg0)
    scale(iea0, rows0)
    fire_scatter(NCH - 1, rows0, sems0)
    wait_dma(rows1, sems1)
    wait_dma(rows0, sems0)

    plsc.subcore_barrier()
    for k in range(5):
        off = s * 640 + k * 128
        pltpu.sync_copy(acc.at[pl.ds(off, 128)],
                        out_hbm.at[c, pl.ds(off, 128)])


_msg_kernel = pl.kernel(
    _msg_body,
    out_type=jax.ShapeDtypeStruct((NC, NPAD, D), _f32),
    mesh=plsc.VectorSubcoreMesh(core_axis_name="c", subcore_axis_name="s"),
    compiler_params=_SC_PARAMS,
    scratch_types=(
        [pltpu.VMEM((NCH, CH), jnp.int32)]
        + [pltpu.VMEM((2, CH), jnp.int32) for _ in range(2)]
        + [pltpu.VMEM((CH, D), _f32) for _ in range(2)]
        + [pltpu.VMEM_SHARED((NPAD, D), _f32)]
        + [pltpu.SemaphoreType.DMA for _ in range(6)]
    ),
)


RB = 400
NBLK = N // RB


def _fused0_body(x_ref, dv_ref, w1_ref, b1_ref, g_ref, bt_ref, w2_ref,
                 b2_ref, o_ref, h1s, sts):
    ph = pl.program_id(0)
    i = pl.program_id(1)

    @pl.when(ph == 0)
    def _():
        @pl.when(i == 0)
        def _():
            sts[...] = jnp.zeros((8, D), _f32)

        a = jnp.dot(x_ref[...], w1_ref[...],
                    preferred_element_type=_f32) + b1_ref[...]
        h1s[pl.ds(i * RB, RB), :] = a
        sts[0:1, :] += jnp.sum(a, axis=0, keepdims=True)
        sts[1:2, :] += jnp.sum(a * a, axis=0, keepdims=True)
        o_ref[...] = a

    @pl.when(ph == 1)
    def _():
        m = sts[0:1, :] * (1.0 / N)
        ex2 = sts[1:2, :] * (1.0 / N)
        var = ex2 - m * m
        rstd = lax.rsqrt(var + 1e-5)
        aa = g_ref[...] * rstd
        cc = bt_ref[...] - m * aa
        t = _leaky(h1s[pl.ds(i * RB, RB), :] * aa + cc)
        o_ref[...] = dv_ref[...] * (
            jnp.dot(t, w2_ref[...], preferred_element_type=_f32)
            + b2_ref[...])


def _fused0(x, dinv_col, w1, b1, gbn, bt, w2, b2):
    return pl.pallas_call(
        _fused0_body,
        grid=(2, NBLK),
        in_specs=[
            pl.BlockSpec((RB, D), lambda p, i: (i * (1 - p), 0)),
            pl.BlockSpec((RB, 1), lambda p, i: (i, 0)),
            pl.BlockSpec((D, D), lambda p, i: (0, 0)),
            pl.BlockSpec((1, D), lambda p, i: (0, 0)),
            pl.BlockSpec((1, D), lambda p, i: (0, 0)),
            pl.BlockSpec((1, D), lambda p, i: (0, 0)),
            pl.BlockSpec((D, D), lambda p, i: (0, 0)),
            pl.BlockSpec((1, D), lambda p, i: (0, 0)),
        ],
        out_specs=pl.BlockSpec((RB, D), lambda p, i: (i, 0)),
        out_shape=jax.ShapeDtypeStruct((N, D), _f32),
        scratch_shapes=[pltpu.VMEM((N, D), _f32), pltpu.VMEM((8, D), _f32)],
    )(x, dinv_col, w1, b1.reshape(1, D), gbn.reshape(1, D),
      bt.reshape(1, D), w2, b2.reshape(1, D))


def _fused1_body(p0_ref, p1_ref, dv_ref, w1_ref, b1_ref, g_ref, bt_ref,
                 w2_ref, b2_ref, o_ref, h1s, sts):
    ph = pl.program_id(0)
    i = pl.program_id(1)

    @pl.when(ph == 0)
    def _():
        @pl.when(i == 0)
        def _():
            sts[...] = jnp.zeros((8, D), _f32)

        x = _leaky(dv_ref[...] * (p0_ref[0] + p1_ref[0]))
        a = jnp.dot(x, w1_ref[...],
                    preferred_element_type=_f32) + b1_ref[...]
        h1s[pl.ds(i * RB, RB), :] = a
        sts[0:1, :] += jnp.sum(a, axis=0, keepdims=True)
        sts[1:2, :] += jnp.sum(a * a, axis=0, keepdims=True)
        o_ref[...] = a

    @pl.when(ph == 1)
    def _():
        m = sts[0:1, :] * (1.0 / N)
        ex2 = sts[1:2, :] * (1.0 / N)
        var = ex2 - m * m
        rstd = lax.rsqrt(var + 1e-5)
        aa = g_ref[...] * rstd
        cc = bt_ref[...] - m * aa
        t = _leaky(h1s[pl.ds(i * RB, RB), :] * aa + cc)
        o_ref[...] = dv_ref[...] * (
            jnp.dot(t, w2_ref[...], preferred_element_type=_f32)
            + b2_ref[...])


def _fused1(parts, dinv_col, w1, b1, gbn, bt, w2, b2):
    return pl.pallas_call(
        _fused1_body,
        grid=(2, NBLK),
        in_specs=[
            pl.BlockSpec((1, RB, D), lambda p, i: (0, i * (1 - p), 0)),
            pl.BlockSpec((1, RB, D), lambda p, i: (1, i * (1 - p), 0)),
            pl.BlockSpec((RB, 1), lambda p, i: (i, 0)),
            pl.BlockSpec((D, D), lambda p, i: (0, 0)),
            pl.BlockSpec((1, D), lambda p, i: (0, 0)),
            pl.BlockSpec((1, D), lambda p, i: (0, 0)),
            pl.BlockSpec((1, D), lambda p, i: (0, 0)),
            pl.BlockSpec((D, D), lambda p, i: (0, 0)),
            pl.BlockSpec((1, D), lambda p, i: (0, 0)),
        ],
        out_specs=pl.BlockSpec((RB, D), lambda p, i: (i, 0)),
        out_shape=jax.ShapeDtypeStruct((N, D), _f32),
        scratch_shapes=[pltpu.VMEM((N, D), _f32), pltpu.VMEM((8, D), _f32)],
    )(parts, parts, dinv_col, w1, b1.reshape(1, D), gbn.reshape(1, D),
      bt.reshape(1, D), w2, b2.reshape(1, D))


def _final_body(q0_ref, q1_ref, dv_ref, bcol_ref, blo_ref, bhi_ref, en_ref,
                ew_ref, eb_ref, eg_ref, ebt_ref, wa_ref, wb_ref, lb1_ref,
                lg_ref, lbt_ref, lw2_ref, lb2_ref, o_ref, pooled):
    i = pl.program_id(0)

    @pl.when(i == 0)
    def _():
        pooled[...] = jnp.full((B, D), -jnp.inf, _f32)

    h = _leaky(dv_ref[...] * (q0_ref[0] + q1_ref[0]))
    bblk = bcol_ref[0]
    blo = blo_ref[i]
    bhi = bhi_ref[i]

    def seg(b, _):
        mask = bblk == b
        vals = jnp.where(mask, h, -jnp.inf)
        m = jnp.max(vals, axis=0, keepdims=True)
        pooled[pl.ds(b, 1)] = jnp.maximum(pooled[pl.ds(b, 1)], m)
        return 0
    lax.fori_loop(blo, bhi + 1, seg, 0)

    @pl.when(i == NBLK - 1)
    def _():
        e0 = jnp.dot(en_ref[...], ew_ref[...], preferred_element_type=_f32) \
            + eb_ref[...]
        m = jnp.mean(e0, axis=0, keepdims=True)
        v = jnp.mean((e0 - m) * (e0 - m), axis=0, keepdims=True)
        e1 = _leaky((e0 - m) * lax.rsqrt(v + 1e-5) * eg_ref[...] + ebt_ref[...])
        z = jnp.dot(pooled[...], wa_ref[...], preferred_element_type=_f32) \
            + jnp.dot(e1, wb_ref[...], preferred_element_type=_f32) \
            + lb1_ref[...]
        zm = jnp.mean(z, axis=0, keepdims=True)
        zv = jnp.mean((z - zm) * (z - zm), axis=0, keepdims=True)
        z2 = _leaky((z - zm) * lax.rsqrt(zv + 1e-5) * lg_ref[...] + lbt_ref[...])
        y = jnp.dot(z2, lw2_ref[...], preferred_element_type=_f32) + lb2_ref[...]
        o_ref[...] = -jnp.maximum(y, 0.0)


def _final(qparts, dinv_col, bcol, blo, bhi, energy, p):
    wa = p['l_w1'][:D, :]
    wb = p['l_w1'][D:, :]
    return pl.pallas_call(
        _final_body,
        grid=(NBLK,),
        in_specs=[
            pl.BlockSpec((1, RB, D), lambda i: (0, i, 0)),
            pl.BlockSpec((1, RB, D), lambda i: (1, i, 0)),
            pl.BlockSpec((RB, 1), lambda i: (i, 0)),
            pl.BlockSpec((1, RB, 1), lambda i: (i, 0, 0)),
            pl.BlockSpec(memory_space=pltpu.SMEM),
            pl.BlockSpec(memory_space=pltpu.SMEM),
            pl.BlockSpec((B, 21), lambda i: (0, 0)),
            pl.BlockSpec((21, 8), lambda i: (0, 0)),
            pl.BlockSpec((1, 8), lambda i: (0, 0)),
            pl.BlockSpec((1, 8), lambda i: (0, 0)),
            pl.BlockSpec((1, 8), lambda i: (0, 0)),
            pl.BlockSpec((D, D), lambda i: (0, 0)),
            pl.BlockSpec((8, D), lambda i: (0, 0)),
            pl.BlockSpec((1, D), lambda i: (0, 0)),
            pl.BlockSpec((1, D), lambda i: (0, 0)),
            pl.BlockSpec((1, D), lambda i: (0, 0)),
            pl.BlockSpec((D, 1), lambda i: (0, 0)),
            pl.BlockSpec((1, 1), lambda i: (0, 0)),
        ],
        out_specs=pl.BlockSpec((B, 1), lambda i: (0, 0)),
        out_shape=jax.ShapeDtypeStruct((B, 1), _f32),
        scratch_shapes=[pltpu.VMEM((B, D), _f32)],
    )(qparts, qparts, dinv_col, bcol, blo, bhi, energy,
      p['e_w'], p['e_b'].reshape(1, 8), p['e_g'].reshape(1, 8),
      p['e_bt'].reshape(1, 8), wa, wb, p['l_b1'].reshape(1, D),
      p['l_g'].reshape(1, D), p['l_bt'].reshape(1, D), p['l_w2'],
      p['l_b2'].reshape(1, 1))


def kernel(x, edge_index, edge_attr, batch, energy, params):
    p = params
    pad = EP - E - N
    loop = jnp.arange(N, dtype=jnp.int32)
    rowp = jnp.concatenate(
        [edge_index[0], loop, jnp.zeros((pad,), jnp.int32)]).reshape(NW * NCH, CH)
    colp = jnp.concatenate(
        [edge_index[1], loop, jnp.zeros((pad,), jnp.int32)]).reshape(NW, NCH, CH)
    eabits = lax.bitcast_convert_type(
        jnp.concatenate([edge_attr, jnp.ones((N,), _f32),
                         jnp.zeros((pad,), _f32)]).reshape(NW * NCH, CH),
        jnp.int32)
    ieab = jnp.stack([rowp, eabits], axis=1)

    hists = _hist_kernel(edge_index[0])
    dinv_col = _dinv_kernel(hists).reshape(N, 1)

    zeros128 = jnp.zeros((128, D), _f32)

    g0 = _fused0(x, dinv_col, p['m0_w1'], p['m0_b1'], p['m0_g'], p['m0_bt'],
                 p['m0_w2'], p['m0_b2'])
    pparts = _msg_kernel(g0, ieab, colp, zeros128)

    g1 = _fused1(pparts, dinv_col, p['m1_w1'], p['m1_b1'],
                 p['m1_g'], p['m1_bt'], p['m1_w2'], p['m1_b2'])
    qparts = _msg_kernel(g1, ieab, colp, zeros128)

    bcol = batch.reshape(NBLK, RB, 1)
    b2d = batch.reshape(NBLK, RB)
    blo = b2d[:, 0]
    bhi = b2d[:, RB - 1]
    return _final(qparts, dinv_col, bcol, blo, bhi, energy, p)


# speedup vs baseline: 1.0088x; 1.0034x over previous
import jax
import jax.numpy as jnp
from jax import lax
from jax.experimental import pallas as pl
from jax.experimental.pallas import tpu as pltpu
from jax.experimental.pallas import tpu_sc as plsc

N = 10000
E = 320000
D = 128
B = 64
NC = 2
NS = 16
NW = NC * NS

EP = 331776
EPT = EP // NW
NPAD = 10240

_f32 = jnp.float32


def _leaky(x):
    return jnp.where(x > 0, x, 0.01 * x)


def _hist_body(row_hbm, out_hbm, idxt_v, hist_v):
    c = lax.axis_index("c")
    s = lax.axis_index("s")
    wid = s * NC + c
    ept = E // NW

    def zbody(i, _):
        hist_v[pl.ds(i * 16, 16)] = jnp.zeros((16,), _f32)
        return 0
    lax.fori_loop(0, N // 16, zbody, 0)

    pltpu.sync_copy(row_hbm.at[pl.ds(wid * ept, ept)], idxt_v)

    ones16 = jnp.ones((16,), _f32)

    def grp(i, _):
        idx16 = idxt_v[pl.ds(i * 16, 16)]
        plsc.addupdate_scatter(hist_v, [idx16], ones16)
        return 0
    lax.fori_loop(0, ept // 16, grp, 0)

    pltpu.sync_copy(hist_v, out_hbm.at[wid])


_SC_PARAMS = pltpu.CompilerParams(needs_layout_passes=False)

_hist_kernel = pl.kernel(
    _hist_body,
    out_type=jax.ShapeDtypeStruct((NW, N), _f32),
    mesh=plsc.VectorSubcoreMesh(core_axis_name="c", subcore_axis_name="s"),
    compiler_params=_SC_PARAMS,
    scratch_types=[
        pltpu.VMEM((E // NW,), jnp.int32),
        pltpu.VMEM((N,), _f32),
    ],
)


def _dinv_body(h_ref, o_ref):
    s = jnp.sum(h_ref[...], axis=0, keepdims=True)
    o_ref[...] = lax.rsqrt(s + 1.0)


def _dinv_kernel(hists):
    return pl.pallas_call(
        _dinv_body,
        out_shape=jax.ShapeDtypeStruct((1, N), _f32),
    )(hists)


CH = 128
NCH = EPT // CH


def _msg_body(g_hbm, ieab, colb, zeros_hbm, out_hbm, colt, iea0, iea1,
              rows0, rows1, acc, semi0, semi1, semg0, semg1, sems0, sems1):
    c = lax.axis_index("c")
    s = lax.axis_index("s")
    wid = s * NC + c
    cbase = wid * NCH

    def stage(ci, ieabuf, sem):
        pltpu.async_copy(ieab.at[cbase + ci], ieabuf, sem)

    def wait_stage(ieabuf, sem):
        pltpu.make_async_copy(ieab.at[0], ieabuf, sem).wait()

    def fire_gather(ieabuf, buf, sem):
        pltpu.async_copy(g_hbm.at[ieabuf.at[0]], buf, sem)

    def wait_dma(buf, sem):
        pltpu.make_async_copy(zeros_hbm, buf, sem).wait()

    def fire_scatter(ci, buf, sem):
        pltpu.async_copy(buf, acc.at[colt.at[ci]], sem, add=True)

    row1sel = jnp.ones((16,), jnp.int32)

    def scale(ieabuf, buf):
        def ebody(e, _):
            eidx = jnp.broadcast_to(e.astype(jnp.int32), (16,))
            cv = plsc.bitcast(plsc.load_gather(ieabuf, [row1sel, eidx]),
                              _f32)
            for j in range(8):
                sl = pl.ds(j * 16, 16)
                buf[e, sl] = buf[e, sl] * cv
            return 0
        lax.fori_loop(0, CH, ebody, 0, unroll=4)

    stage(0, iea0, semi0)
    stage(1, iea1, semi1)
    pltpu.sync_copy(colb.at[wid], colt)
    wait_stage(iea0, semi0)
    fire_gather(iea0, rows0, semg0)
    for k in range(5):
        pltpu.sync_copy(zeros_hbm, acc.at[pl.ds(s * 640 + k * 128, 128)])
    plsc.subcore_barrier()

    def pair(c0, first):
        c1 = c0 + 1
        wait_stage(iea1, semi1)
        if not first:
            wait_dma(rows1, sems1)
        fire_gather(iea1, rows1, semg1)
        wait_dma(rows0, semg0)
        scale(iea0, rows0)
        fire_scatter(c0, rows0, sems0)
        stage(c0 + 2, iea0, semi0)
        wait_dma(rows1, semg1)
        scale(iea1, rows1)
        fire_scatter(c1, rows1, sems1)
        stage(jnp.minimum(c0 + 3, NCH - 1), iea1, semi1)
        wait_stage(iea0, semi0)
        wait_dma(rows0, sems0)
        fire_gather(iea0, rows0, semg0)

    pair(0, True)

    def pbody(i2, _):
        pair(2 * i2, False)
        return 0
    lax.fori_loop(1, (NCH - 1) // 2, pbody, 0)

    wait_stage(iea1, semi1)
    wait_dma(rows0, semg0)
    scale(iea0, rows0)
    fire_scatter(NCH - 1, rows0, sems0)
    wait_dma(rows1, sems1)
    wait_dma(rows0, sems0)

    plsc.subcore_barrier()
    for k in range(5):
        off = s * 640 + k * 128
        pltpu.sync_copy(acc.at[pl.ds(off, 128)],
                        out_hbm.at[c, pl.ds(off, 128)])


_msg_kernel = pl.kernel(
    _msg_body,
    out_type=jax.ShapeDtypeStruct((NC, NPAD, D), _f32),
    mesh=plsc.VectorSubcoreMesh(core_axis_name="c", subcore_axis_name="s"),
    compiler_params=_SC_PARAMS,
    scratch_types=(
        [pltpu.VMEM((NCH, CH), jnp.int32)]
        + [pltpu.VMEM((2, CH), jnp.int32) for _ in range(2)]
        + [pltpu.VMEM((CH, D), _f32) for _ in range(2)]
        + [pltpu.VMEM_SHARED((NPAD, D), _f32)]
        + [pltpu.SemaphoreType.DMA for _ in range(6)]
    ),
)


RB = 400
NBLK = N // RB


def _fused0_body(x_ref, dv_ref, w1_ref, b1_ref, g_ref, bt_ref, w2_ref,
                 b2_ref, o_ref, h1s, sts):
    ph = pl.program_id(0)
    i = pl.program_id(1)

    @pl.when(ph == 0)
    def _():
        @pl.when(i == 0)
        def _():
            sts[...] = jnp.zeros((8, D), _f32)

        a = jnp.dot(x_ref[...], w1_ref[...],
                    preferred_element_type=_f32) + b1_ref[...]
        h1s[pl.ds(i * RB, RB), :] = a
        sts[0:1, :] += jnp.sum(a, axis=0, keepdims=True)
        sts[1:2, :] += jnp.sum(a * a, axis=0, keepdims=True)
        o_ref[...] = a

    @pl.when(ph == 1)
    def _():
        m = sts[0:1, :] * (1.0 / N)
        ex2 = sts[1:2, :] * (1.0 / N)
        var = ex2 - m * m
        rstd = lax.rsqrt(var + 1e-5)
        aa = g_ref[...] * rstd
        cc = bt_ref[...] - m * aa
        t = _leaky(h1s[pl.ds(i * RB, RB), :] * aa + cc)
        o_ref[...] = dv_ref[...] * (
            jnp.dot(t, w2_ref[...], preferred_element_type=_f32)
            + b2_ref[...])


def _fused0(x, dinv_col, w1, b1, gbn, bt, w2, b2):
    return pl.pallas_call(
        _fused0_body,
        grid=(2, NBLK),
        in_specs=[
            pl.BlockSpec((RB, D), lambda p, i: (i * (1 - p), 0)),
            pl.BlockSpec((RB, 1), lambda p, i: (i, 0)),
            pl.BlockSpec((D, D), lambda p, i: (0, 0)),
            pl.BlockSpec((1, D), lambda p, i: (0, 0)),
            pl.BlockSpec((1, D), lambda p, i: (0, 0)),
            pl.BlockSpec((1, D), lambda p, i: (0, 0)),
            pl.BlockSpec((D, D), lambda p, i: (0, 0)),
            pl.BlockSpec((1, D), lambda p, i: (0, 0)),
        ],
        out_specs=pl.BlockSpec((RB, D), lambda p, i: (i, 0)),
        out_shape=jax.ShapeDtypeStruct((N, D), _f32),
        scratch_shapes=[pltpu.VMEM((N, D), _f32), pltpu.VMEM((8, D), _f32)],
    )(x, dinv_col, w1, b1.reshape(1, D), gbn.reshape(1, D),
      bt.reshape(1, D), w2, b2.reshape(1, D))


def _fused1_body(p0_ref, p1_ref, dv_ref, w1_ref, b1_ref, g_ref, bt_ref,
                 w2_ref, b2_ref, o_ref, h1s, sts):
    ph = pl.program_id(0)
    i = pl.program_id(1)

    @pl.when(ph == 0)
    def _():
        @pl.when(i == 0)
        def _():
            sts[...] = jnp.zeros((8, D), _f32)

        x = _leaky(dv_ref[...] * (p0_ref[0] + p1_ref[0]))
        a = jnp.dot(x, w1_ref[...],
                    preferred_element_type=_f32) + b1_ref[...]
        h1s[pl.ds(i * RB, RB), :] = a
        sts[0:1, :] += jnp.sum(a, axis=0, keepdims=True)
        sts[1:2, :] += jnp.sum(a * a, axis=0, keepdims=True)
        o_ref[...] = a

    @pl.when(ph == 1)
    def _():
        m = sts[0:1, :] * (1.0 / N)
        ex2 = sts[1:2, :] * (1.0 / N)
        var = ex2 - m * m
        rstd = lax.rsqrt(var + 1e-5)
        aa = g_ref[...] * rstd
        cc = bt_ref[...] - m * aa
        t = _leaky(h1s[pl.ds(i * RB, RB), :] * aa + cc)
        o_ref[...] = dv_ref[...] * (
            jnp.dot(t, w2_ref[...], preferred_element_type=_f32)
            + b2_ref[...])


def _fused1(parts, dinv_col, w1, b1, gbn, bt, w2, b2):
    return pl.pallas_call(
        _fused1_body,
        grid=(2, NBLK),
        in_specs=[
            pl.BlockSpec((1, RB, D), lambda p, i: (0, i * (1 - p), 0)),
            pl.BlockSpec((1, RB, D), lambda p, i: (1, i * (1 - p), 0)),
            pl.BlockSpec((RB, 1), lambda p, i: (i, 0)),
            pl.BlockSpec((D, D), lambda p, i: (0, 0)),
            pl.BlockSpec((1, D), lambda p, i: (0, 0)),
            pl.BlockSpec((1, D), lambda p, i: (0, 0)),
            pl.BlockSpec((1, D), lambda p, i: (0, 0)),
            pl.BlockSpec((D, D), lambda p, i: (0, 0)),
            pl.BlockSpec((1, D), lambda p, i: (0, 0)),
        ],
        out_specs=pl.BlockSpec((RB, D), lambda p, i: (i, 0)),
        out_shape=jax.ShapeDtypeStruct((N, D), _f32),
        scratch_shapes=[pltpu.VMEM((N, D), _f32), pltpu.VMEM((8, D), _f32)],
    )(parts, parts, dinv_col, w1, b1.reshape(1, D), gbn.reshape(1, D),
      bt.reshape(1, D), w2, b2.reshape(1, D))


def _final_body(q0_ref, q1_ref, dv_ref, bcol_ref, blo_ref, bhi_ref, en_ref,
                ew_ref, eb_ref, eg_ref, ebt_ref, wa_ref, wb_ref, lb1_ref,
                lg_ref, lbt_ref, lw2_ref, lb2_ref, o_ref, pooled):
    i = pl.program_id(0)

    @pl.when(i == 0)
    def _():
        pooled[...] = jnp.full((B, D), -jnp.inf, _f32)

    h = _leaky(dv_ref[...] * (q0_ref[0] + q1_ref[0]))
    bblk = bcol_ref[0]
    blo = blo_ref[i]
    bhi = bhi_ref[i]

    def seg(b, _):
        mask = bblk == b
        vals = jnp.where(mask, h, -jnp.inf)
        m = jnp.max(vals, axis=0, keepdims=True)
        pooled[pl.ds(b, 1)] = jnp.maximum(pooled[pl.ds(b, 1)], m)
        return 0
    lax.fori_loop(blo, bhi + 1, seg, 0)

    @pl.when(i == NBLK - 1)
    def _():
        e0 = jnp.dot(en_ref[...], ew_ref[...], preferred_element_type=_f32) \
            + eb_ref[...]
        m = jnp.mean(e0, axis=0, keepdims=True)
        v = jnp.mean((e0 - m) * (e0 - m), axis=0, keepdims=True)
        e1 = _leaky((e0 - m) * lax.rsqrt(v + 1e-5) * eg_ref[...] + ebt_ref[...])
        z = jnp.dot(pooled[...], wa_ref[...], preferred_element_type=_f32) \
            + jnp.dot(e1, wb_ref[...], preferred_element_type=_f32) \
            + lb1_ref[...]
        zm = jnp.mean(z, axis=0, keepdims=True)
        zv = jnp.mean((z - zm) * (z - zm), axis=0, keepdims=True)
        z2 = _leaky((z - zm) * lax.rsqrt(zv + 1e-5) * lg_ref[...] + lbt_ref[...])
        y = jnp.dot(z2, lw2_ref[...], preferred_element_type=_f32) + lb2_ref[...]
        o_ref[...] = -jnp.maximum(y, 0.0)


def _final(qparts, dinv_col, bcol, blo, bhi, energy, p):
    wa = p['l_w1'][:D, :]
    wb = p['l_w1'][D:, :]
    return pl.pallas_call(
        _final_body,
        grid=(NBLK,),
        in_specs=[
            pl.BlockSpec((1, RB, D), lambda i: (0, i, 0)),
            pl.BlockSpec((1, RB, D), lambda i: (1, i, 0)),
            pl.BlockSpec((RB, 1), lambda i: (i, 0)),
            pl.BlockSpec((1, RB, 1), lambda i: (i, 0, 0)),
            pl.BlockSpec(memory_space=pltpu.SMEM),
            pl.BlockSpec(memory_space=pltpu.SMEM),
            pl.BlockSpec((B, 21), lambda i: (0, 0)),
            pl.BlockSpec((21, 8), lambda i: (0, 0)),
            pl.BlockSpec((1, 8), lambda i: (0, 0)),
            pl.BlockSpec((1, 8), lambda i: (0, 0)),
            pl.BlockSpec((1, 8), lambda i: (0, 0)),
            pl.BlockSpec((D, D), lambda i: (0, 0)),
            pl.BlockSpec((8, D), lambda i: (0, 0)),
            pl.BlockSpec((1, D), lambda i: (0, 0)),
            pl.BlockSpec((1, D), lambda i: (0, 0)),
            pl.BlockSpec((1, D), lambda i: (0, 0)),
            pl.BlockSpec((D, 1), lambda i: (0, 0)),
            pl.BlockSpec((1, 1), lambda i: (0, 0)),
        ],
        out_specs=pl.BlockSpec((B, 1), lambda i: (0, 0)),
        out_shape=jax.ShapeDtypeStruct((B, 1), _f32),
        scratch_shapes=[pltpu.VMEM((B, D), _f32)],
    )(qparts, qparts, dinv_col, bcol, blo, bhi, energy,
      p['e_w'], p['e_b'].reshape(1, 8), p['e_g'].reshape(1, 8),
      p['e_bt'].reshape(1, 8), wa, wb, p['l_b1'].reshape(1, D),
      p['l_g'].reshape(1, D), p['l_bt'].reshape(1, D), p['l_w2'],
      p['l_b2'].reshape(1, 1))


def kernel(x, edge_index, edge_attr, batch, energy, params):
    p = params
    pad = EP - E - N
    loop = jnp.arange(N, dtype=jnp.int32)
    rowp = jnp.concatenate(
        [edge_index[0], loop, jnp.zeros((pad,), jnp.int32)]).reshape(NW * NCH, CH)
    colp = jnp.concatenate(
        [edge_index[1], loop, jnp.zeros((pad,), jnp.int32)]).reshape(NW, NCH, CH)
    eabits = lax.bitcast_convert_type(
        jnp.concatenate([edge_attr, jnp.ones((N,), _f32),
                         jnp.zeros((pad,), _f32)]).reshape(NW * NCH, CH),
        jnp.int32)
    ieab = jnp.stack([rowp, eabits], axis=1)

    hists = _hist_kernel(edge_index[0])
    dinv_col = _dinv_kernel(hists).reshape(N, 1)

    zeros128 = jnp.zeros((128, D), _f32)

    g0 = _fused0(x, dinv_col, p['m0_w1'], p['m0_b1'], p['m0_g'], p['m0_bt'],
                 p['m0_w2'], p['m0_b2'])
    pparts = _msg_kernel(g0, ieab, colp, zeros128)

    g1 = _fused1(pparts, dinv_col, p['m1_w1'], p['m1_b1'],
                 p['m1_g'], p['m1_bt'], p['m1_w2'], p['m1_b2'])
    qparts = _msg_kernel(g1, ieab, colp, zeros128)

    bcol = batch.reshape(NBLK, RB, 1)
    b2d = batch.reshape(NBLK, RB)
    blo = b2d[:, 0]
    bhi = b2d[:, RB - 1]
    return _final(qparts, dinv_col, bcol, blo, bhi, energy, p)
